# baseline (speedup 1.0000x reference)
import functools

import jax
import jax.numpy as jnp
from jax.experimental import pallas as pl
from jax.experimental.pallas import tpu as pltpu

_DECAY = 0.3
_THRESHOLD = 0.9
_BLOCK = 5000


def _body(n_steps, sf_ref, tf_ref, ys_ref, yt_ref, scT_ref, tcT_ref, out_ref,
          ssumT, tsumT, scnt, tcnt):
    i = pl.program_id(0)

    @pl.when(i == 0)
    def _init():
        ssumT[...] = jnp.zeros_like(ssumT)
        tsumT[...] = jnp.zeros_like(tsumT)
        scnt[...] = jnp.zeros_like(scnt)
        tcnt[...] = jnp.zeros_like(tcnt)

    def accum(y, f, sumT_ref, cnt_ref):
        b, c = y.shape
        mx = jnp.max(y, axis=1, keepdims=True)
        iota_s = jax.lax.broadcasted_iota(jnp.int32, (b, c), 1)
        iota = iota_s.astype(jnp.float32)
        idx = jnp.min(jnp.where(y == mx, iota, float(c)), axis=1, keepdims=True)
        onehot = jnp.where(iota_s == idx.astype(jnp.int32), 1.0, 0.0)
        sel = jnp.where(mx > _THRESHOLD, mx, 0.0)
        sumT_ref[...] += jax.lax.dot_general(
            f, onehot * sel, (((0,), (0,)), ((), ())),
            preferred_element_type=jnp.float32)
        cnt_ref[...] += jax.lax.dot_general(
            jnp.ones((b, 1), jnp.float32), onehot, (((0,), (0,)), ((), ())),
            preferred_element_type=jnp.float32)

    accum(ys_ref[...], sf_ref[...], ssumT, scnt)
    accum(yt_ref[...], tf_ref[...], tsumT, tcnt)

    @pl.when(i == n_steps - 1)
    def _finish():
        sn = jnp.maximum(scnt[...], 1.0)
        tn = jnp.maximum(tcnt[...], 1.0)
        diff = ((1.0 - _DECAY) * (scT_ref[...] - tcT_ref[...])
                + _DECAY * (ssumT[...] / sn - tsumT[...] / tn))
        out_ref[...] = (jnp.sum(diff * diff) / float(diff.size)).reshape(1, 1)


def kernel(s_feature, t_feature, y_s, y_t, s_centroid, t_centroid):
    n, d = s_feature.shape
    c = y_s.shape[1]
    block = _BLOCK
    n_steps = n // block
    assert n_steps * block == n

    row_spec = lambda w: pl.BlockSpec((block, w), lambda i: (i, 0))
    fixed_spec = pl.BlockSpec((d, c), lambda i: (0, 0))
    out = pl.pallas_call(
        functools.partial(_body, n_steps),
        grid=(n_steps,),
        in_specs=[row_spec(d), row_spec(d), row_spec(c), row_spec(c),
                  fixed_spec, fixed_spec],
        out_specs=pl.BlockSpec((1, 1), lambda i: (0, 0)),
        out_shape=jax.ShapeDtypeStruct((1, 1), jnp.float32),
        scratch_shapes=[
            pltpu.VMEM((d, c), jnp.float32),
            pltpu.VMEM((d, c), jnp.float32),
            pltpu.VMEM((1, c), jnp.float32),
            pltpu.VMEM((1, c), jnp.float32),
        ],
    )(s_feature, t_feature, y_s, y_t,
      s_centroid.T, t_centroid.T)
    return out[0, 0]


# speedup vs baseline: 1.0088x; 1.0088x over previous
import functools

import jax
import jax.numpy as jnp
from jax.experimental import pallas as pl
from jax.experimental.pallas import tpu as pltpu

_DECAY = 0.3
_THRESHOLD = 0.9
_BLOCK = 5000


def _body(n_steps, sf_ref, tf_ref, ys_ref, yt_ref, scT_ref, tcT_ref, out_ref,
          ssumT, tsumT, scnt, tcnt):
    i = pl.program_id(0)

    @pl.when(i == 0)
    def _init():
        ssumT[...] = jnp.zeros_like(ssumT)
        tsumT[...] = jnp.zeros_like(tsumT)
        scnt[...] = jnp.zeros_like(scnt)
        tcnt[...] = jnp.zeros_like(tcnt)

    def accum(y, f, sumT_ref, cnt_ref):
        b, c = y.shape
        mx = jnp.max(y, axis=1, keepdims=True)
        iota = jax.lax.broadcasted_iota(jnp.int32, (b, c), 1).astype(jnp.float32)
        idx = jnp.min(jnp.where(y == mx, iota, float(c)), axis=1, keepdims=True)
        onehot = jnp.where(iota == idx, 1.0, 0.0)
        sel = jnp.where(mx > _THRESHOLD, mx, 0.0)
        sumT_ref[...] += jax.lax.dot_general(
            f, onehot * sel, (((0,), (0,)), ((), ())),
            preferred_element_type=jnp.float32)
        cnt_ref[...] += jnp.sum(onehot, axis=0, keepdims=True)

    accum(ys_ref[...], sf_ref[...], ssumT, scnt)
    accum(yt_ref[...], tf_ref[...], tsumT, tcnt)

    @pl.when(i == n_steps - 1)
    def _finish():
        sn = jnp.maximum(scnt[...], 1.0)
        tn = jnp.maximum(tcnt[...], 1.0)
        diff = ((1.0 - _DECAY) * (scT_ref[...] - tcT_ref[...])
                + _DECAY * (ssumT[...] / sn - tsumT[...] / tn))
        out_ref[...] = (jnp.sum(diff * diff) / float(diff.size)).reshape(1, 1)


def kernel(s_feature, t_feature, y_s, y_t, s_centroid, t_centroid):
    n, d = s_feature.shape
    c = y_s.shape[1]
    block = _BLOCK
    n_steps = n // block
    assert n_steps * block == n

    row_spec = lambda w: pl.BlockSpec((block, w), lambda i: (i, 0))
    fixed_spec = pl.BlockSpec((d, c), lambda i: (0, 0))
    out = pl.pallas_call(
        functools.partial(_body, n_steps),
        grid=(n_steps,),
        in_specs=[row_spec(d), row_spec(d), row_spec(c), row_spec(c),
                  fixed_spec, fixed_spec],
        out_specs=pl.BlockSpec((1, 1), lambda i: (0, 0)),
        out_shape=jax.ShapeDtypeStruct((1, 1), jnp.float32),
        scratch_shapes=[
            pltpu.VMEM((d, c), jnp.float32),
            pltpu.VMEM((d, c), jnp.float32),
            pltpu.VMEM((1, c), jnp.float32),
            pltpu.VMEM((1, c), jnp.float32),
        ],
    )(s_feature, t_feature, y_s, y_t,
      s_centroid.T, t_centroid.T)
    return out[0, 0]
